# baseline (speedup 1.0000x reference)
import functools

import jax
import jax.numpy as jnp
from jax import lax
from jax.experimental import pallas as pl
from jax.experimental.pallas import tpu as pltpu
from jax.experimental.pallas import tpu_sc as plsc

NC = 2
NS = 16
CH = 128
SG = 8


def _row_split(n):
    rpt = (n // NS) // 8 * 8
    last = n - (NS - 1) * rpt
    assert last % 8 == 0 and 0 < last
    return rpt, last


def _span(sid, copy, rpt, last):
    @pl.when(sid < NS - 1)
    def _():
        copy(rpt)

    @pl.when(sid == NS - 1)
    def _():
        copy(last)


def _sc_mesh():
    return plsc.VectorSubcoreMesh(core_axis_name="c", subcore_axis_name="s",
                                  num_cores=NC, num_subcores=NS)


def _make_deg_kernel(n, ep, w):
    tiles = NC * NS
    ept = ep // tiles
    nch = ept // CH
    assert ept * tiles == ep and nch * CH == ept
    rpt, last = _row_split(n)

    @functools.partial(
        pl.kernel,
        out_type=jax.ShapeDtypeStruct((NC, n, w), jnp.float32),
        mesh=_sc_mesh(),
        scratch_types=[
            pltpu.VMEM_SHARED((n + 8, w), jnp.float32),
            pltpu.VMEM((CH, w), jnp.float32),
            pltpu.VMEM((nch, CH), jnp.int32),
        ],
    )
    def deg_kernel(dst_hbm, zeros_hbm, ones_hbm, out_hbm, acc, ones_v, didx):
        cid = lax.axis_index("c")
        sid = lax.axis_index("s")
        tid = cid * NS + sid
        row0 = pl.multiple_of(sid * rpt, 8)
        pltpu.sync_copy(ones_hbm, ones_v)
        pltpu.sync_copy(dst_hbm.at[tid], didx)
        _span(sid, lambda nr: pltpu.sync_copy(
            zeros_hbm.at[pl.ds(row0, nr)], acc.at[pl.ds(row0, nr)]),
            rpt, last + 8)
        plsc.subcore_barrier()

        def body(j, _):
            pltpu.sync_copy(ones_v, acc.at[didx.at[j]], add=True)
            return 0
        lax.fori_loop(0, nch, body, 0)
        plsc.subcore_barrier()
        _span(sid, lambda nr: pltpu.sync_copy(
            acc.at[pl.ds(row0, nr)], out_hbm.at[cid, pl.ds(row0, nr)]),
            rpt, last)

    return deg_kernel


def _make_seg3_kernel(n, ep, h):
    tiles = NC * NS
    ept = ep // tiles
    nch = ept // CH
    nsg = nch // SG
    nsgp = nsg // 2
    assert ept * tiles == ep and nch * CH == ept and nsgp * 2 * SG == nch
    rpt, last = _row_split(n)

    @functools.partial(
        pl.kernel,
        out_type=[jax.ShapeDtypeStruct((NC, n, h), jnp.float32)] * 3,
        mesh=_sc_mesh(),
        scratch_types=[
            pltpu.VMEM_SHARED((n + 8, h), jnp.float32),
            pltpu.VMEM((2, CH, h), jnp.float32),
            pltpu.VMEM((2, SG, CH), jnp.int32),
            pltpu.VMEM((2, SG, CH), jnp.int32),
            [pltpu.SemaphoreType.DMA] * 2,
            [pltpu.SemaphoreType.DMA] * 4,
        ],
    )
    def seg3_kernel(src_hbm, dst_hbm, x_hbm, s_hbm, t_hbm, zeros_hbm,
                    ox_hbm, os_hbm, ot_hbm,
                    acc, rows, sidx, didx, rsem, isem):
        cid = lax.axis_index("c")
        sid = lax.axis_index("s")
        tid = cid * NS + sid
        row0 = pl.multiple_of(sid * rpt, 8)

        def one_pass(table_hbm, out_hbm):
            _span(sid, lambda nr: pltpu.sync_copy(
                zeros_hbm.at[pl.ds(row0, nr)], acc.at[pl.ds(row0, nr)]),
                rpt, last + 8)
            plsc.subcore_barrier()

            def gather(iset, irow, b):
                pltpu.async_copy(table_hbm.at[sidx.at[iset, irow]],
                                 rows.at[b], rsem[b])

            def scatter(iset, irow, b):
                pltpu.async_copy(rows.at[b], acc.at[didx.at[iset, irow]],
                                 rsem[b], add=True)

            def rwait(b):
                pltpu.make_async_copy(zeros_hbm.at[pl.ds(0, CH)],
                                      rows.at[b], rsem[b]).wait()

            def ifetch(iset, sg):
                pltpu.async_copy(src_hbm.at[tid, sg], sidx.at[iset],
                                 isem[2 * iset])
                pltpu.async_copy(dst_hbm.at[tid, sg], didx.at[iset],
                                 isem[2 * iset + 1])

            def iwait(iset):
                pltpu.make_async_copy(src_hbm.at[tid, 0], sidx.at[iset],
                                      isem[2 * iset]).wait()
                pltpu.make_async_copy(dst_hbm.at[tid, 0], didx.at[iset],
                                      isem[2 * iset + 1]).wait()

            pltpu.sync_copy(src_hbm.at[tid, 0], sidx.at[0])
            pltpu.sync_copy(dst_hbm.at[tid, 0], didx.at[0])

            def pairbody(p, _):
                for cc in range(2 * SG):
                    b = cc % 2
                    pb = 1 - b
                    iset, irow = cc // SG, cc % SG
                    if cc == 0:
                        @pl.when(p > 0)
                        def _():
                            iwait(0)
                    if cc == SG:
                        iwait(1)
                    if cc <= 1:
                        @pl.when(p > 0)
                        def _():
                            rwait(b)
                    else:
                        rwait(b)
                    gather(iset, irow, b)
                    if cc == 0:
                        @pl.when(p > 0)
                        def _():
                            rwait(pb)
                            scatter(1, SG - 1, pb)
                    else:
                        rwait(pb)
                        scatter((cc - 1) // SG, (cc - 1) % SG, pb)
                    if cc == 2:
                        ifetch(1, 2 * p + 1)
                    if cc == SG + 2:
                        @pl.when(p < nsgp - 1)
                        def _():
                            ifetch(0, 2 * p + 2)
                return 0
            lax.fori_loop(0, nsgp, pairbody, 0)
            rwait(1)
            scatter(1, SG - 1, 1)
            rwait(0)
            rwait(1)
            plsc.subcore_barrier()
            _span(sid, lambda nr: pltpu.sync_copy(
                acc.at[pl.ds(row0, nr)], out_hbm.at[cid, pl.ds(row0, nr)]),
                rpt, last)
            plsc.subcore_barrier()

        one_pass(x_hbm, ox_hbm)
        one_pass(s_hbm, os_hbm)
        one_pass(t_hbm, ot_hbm)

    return seg3_kernel


_R = 1000


def _tc_init_kernel(sraw_ref, wemb_ref, bemb_ref, deg_ref,
                    s0_ref, t0_ref, dinv_ref):
    deg = deg_ref[0, :, 0:1] + deg_ref[1, :, 0:1] + 1.0
    dinv = lax.rsqrt(jnp.maximum(deg, 1e-12))
    dinvb = jnp.broadcast_to(dinv, (deg.shape[0], s0_ref.shape[-1]))
    s0 = jnp.dot(sraw_ref[...], wemb_ref[...],
                 preferred_element_type=jnp.float32) + bemb_ref[...]
    s0_ref[...] = s0
    t0_ref[...] = dinvb * s0
    dinv_ref[...] = dinvb


def _tc_layer_kernel(x_ref, s_ref, ax_ref, as_ref, at_ref, dinv_ref,
                     w1x_ref, w1y_ref, b1_ref, w2_ref, b2_ref,
                     wg_ref, bg_ref,
                     xo_ref, so_ref, to_ref):
    dinv = dinv_ref[...]
    gx = x_ref[...] + ax_ref[0] + ax_ref[1]
    gs = s_ref[...] + as_ref[0] + as_ref[1]
    h = jnp.dot(gx, w1x_ref[...], preferred_element_type=jnp.float32)
    h = h + jnp.dot(gs, w1y_ref[...], preferred_element_type=jnp.float32)
    h = jnp.maximum(h + b1_ref[...], 0.0)
    xo = jnp.dot(h, w2_ref[...], preferred_element_type=jnp.float32)
    xo_ref[...] = jnp.maximum(xo + b2_ref[...], 0.0)
    u = dinv * (at_ref[0] + at_ref[1]) + dinv * dinv * s_ref[...]
    so = jnp.tanh(jnp.dot(u, wg_ref[...],
                          preferred_element_type=jnp.float32) + bg_ref[...])
    so_ref[...] = so
    to_ref[...] = dinv * so


def _tc_final_kernel(x_ref, s_ref, wx_ref, wy_ref, b_ref, out_ref):
    o = jnp.dot(x_ref[...], wx_ref[...], preferred_element_type=jnp.float32)
    o = o + jnp.dot(s_ref[...], wy_ref[...], preferred_element_type=jnp.float32)
    out_ref[...] = o + b_ref[...]


def _rows(r, w):
    return pl.BlockSpec((r, w), lambda i: (i, 0))


def _parts(r, w):
    return pl.BlockSpec((NC, r, w), lambda i: (0, i, 0))


def _full(shape):
    return pl.BlockSpec(shape, lambda i: tuple(0 for _ in shape))


def kernel(x, s, edge_index, W_emb, b_emb, gin_W1, gin_b1, gin_W2, gin_b2,
           gcn_W, gcn_b, Whp_W, Whp_b):
    n, h = x.shape
    nse = s.shape[1]
    e = edge_index.shape[1]
    nl = gin_W1.shape[0]
    tiles = NC * NS
    unit = CH * 2 * SG
    ept = -(-e // (tiles * unit)) * unit
    ep = ept * tiles
    nch = ept // CH
    src_p = jnp.concatenate([edge_index[0], jnp.zeros((ep - e,), jnp.int32)])
    dst_p = jnp.concatenate([edge_index[1], jnp.full((ep - e,), n, jnp.int32)])
    src4 = src_p.reshape(tiles, nch // SG, SG, CH)
    dst4 = dst_p.reshape(tiles, nch // SG, SG, CH)
    dst3 = dst_p.reshape(tiles, nch, CH)
    zeros_rows = jnp.zeros((n + 8, h), jnp.float32)
    ones_chunk = jnp.ones((CH, h), jnp.float32)

    deg_parts = _make_deg_kernel(n, ep, h)(dst3, zeros_rows, ones_chunk)
    seg3 = _make_seg3_kernel(n, ep, h)

    grid = (n // _R,)
    s0, t0, dinvb = pl.pallas_call(
        _tc_init_kernel,
        grid=grid,
        in_specs=[_rows(_R, nse), _full((nse, h)), _full((1, h)),
                  _parts(_R, h)],
        out_specs=[_rows(_R, h)] * 3,
        out_shape=[jax.ShapeDtypeStruct((n, h), jnp.float32)] * 3,
    )(s, W_emb, b_emb.reshape(1, h), deg_parts)

    xs, ss, ts = x, s0, t0
    for l in range(nl):
        ax, as_, at = seg3(src4, dst4, xs, ss, ts, zeros_rows)
        xs, ss, ts = pl.pallas_call(
            _tc_layer_kernel,
            grid=grid,
            in_specs=[_rows(_R, h), _rows(_R, h),
                      _parts(_R, h), _parts(_R, h), _parts(_R, h),
                      _rows(_R, h),
                      _full((h, h)), _full((h, h)), _full((1, h)),
                      _full((h, h)), _full((1, h)),
                      _full((h, h)), _full((1, h))],
            out_specs=[_rows(_R, h)] * 3,
            out_shape=[jax.ShapeDtypeStruct((n, h), jnp.float32)] * 3,
        )(xs, ss, ax, as_, at, dinvb,
          gin_W1[l, :h], gin_W1[l, h:], gin_b1[l].reshape(1, h),
          gin_W2[l], gin_b2[l].reshape(1, h),
          gcn_W[l], gcn_b[l].reshape(1, h))

    out = pl.pallas_call(
        _tc_final_kernel,
        grid=grid,
        in_specs=[_rows(_R, h), _rows(_R, h),
                  _full((h, h)), _full((h, h)), _full((1, h))],
        out_specs=_rows(_R, h),
        out_shape=jax.ShapeDtypeStruct((n, h), jnp.float32),
    )(xs, ss, Whp_W[:h], Whp_W[h:], Whp_b.reshape(1, h))
    return out
